# baseline (speedup 1.0000x reference)
import math
import numpy as np
import jax
import jax.numpy as jnp
from jax.experimental import pallas as pl
from jax.experimental.pallas import tpu as pltpu

_BINS = 10
_N_BLOCKS = 6
_MIN_BW = 1e-3
_MIN_BH = 1e-3
_MIN_D = 1e-3
_TILE = 2048
_PREC = jax.lax.Precision.HIGHEST


def _build_mask_table():
    bits = np.array([[int(b) for b in np.binary_repr(i, 3)] for i in range(8)])
    m = bits[:, ::-1].astype(bool).T
    m = np.repeat(m, 2, axis=0)
    m[1::2] = ~m[1::2]
    return m


_MASK_TABLE = _build_mask_table()

_PERM = ([31 * d + b for b in range(_BINS) for d in range(4)]
         + [31 * d + _BINS + b for b in range(_BINS) for d in range(4)]
         + [31 * d + 2 * _BINS + k for k in range(_BINS + 1) for d in range(4)])


def _ssum40(a):
    s = a[0:8] + a[8:16] + a[16:24] + a[24:32] + a[32:40]
    return s[0:4] + s[4:8]


def _ssum44(a):
    s = a[0:8] + a[8:16] + a[16:24] + a[24:32] + a[32:40]
    return s[0:4] + s[4:8] + a[40:44]


def _tile_rows(a, n):
    a8 = jnp.concatenate([a, a], axis=0)
    reps = [a8] * (n // 2)
    if n % 2:
        reps.append(a)
    return jnp.concatenate(reps, axis=0)


def _softplus(v):
    return jnp.maximum(v, 0.0) + jnp.log1p(jnp.exp(-jnp.abs(v)))


def _bsplit(a):
    hi = a.astype(jnp.bfloat16)
    lo = (a - hi.astype(jnp.float32)).astype(jnp.bfloat16)
    return hi, lo


def _dot3(whi, wlo, act):
    ahi, alo = _bsplit(act)
    r = jnp.dot(whi, ahi, preferred_element_type=jnp.float32)
    r = r + jnp.dot(whi, alo, preferred_element_type=jnp.float32)
    r = r + jnp.dot(wlo, ahi, preferred_element_type=jnp.float32)
    return r


def _flow_body(xT_ref, cT_ref, W1hi_ref, W1lo_ref, b1_ref,
               W2hi_ref, W2lo_ref, b2_ref, W3hi_ref, W3lo_ref, b3_ref,
               xo_ref, jac_ref):
    T = xT_ref.shape[1]
    TH = T // 2

    iota40b = (jax.lax.broadcasted_iota(jnp.int32, (40, TH), 0)
               // 4).astype(jnp.float32)
    kvec36 = ((jax.lax.broadcasted_iota(jnp.int32, (36, TH), 0)
               // 4).astype(jnp.float32) + 1.0) * _MIN_BW
    rr = jax.lax.broadcasted_iota(jnp.int32, (36, 40), 0)
    cc = jax.lax.broadcasted_iota(jnp.int32, (36, 40), 1)
    mstrict = (((rr % 4) == (cc % 4)) & ((cc // 4) < (rr // 4 + 1))
               ).astype(jnp.bfloat16)

    for h in range(2):
        lo, hi = h * TH, (h + 1) * TH
        xT = xT_ref[:, lo:hi]
        cT = cT_ref[:, lo:hi]
        xT, jac4 = _flow_half(xT, cT, iota40b, kvec36, mstrict,
                              W1hi_ref, W1lo_ref, b1_ref, W2hi_ref, W2lo_ref,
                              b2_ref, W3hi_ref, W3lo_ref, b3_ref)
        xo_ref[:, lo:hi] = xT
        jac_ref[:, lo:hi] = jnp.sum(jac4, axis=0, keepdims=True)


def _flow_half(xT, cT, iota40b, kvec36, mstrict,
               W1hi_ref, W1lo_ref, b1_ref, W2hi_ref, W2lo_ref, b2_ref,
               W3hi_ref, W3lo_ref, b3_ref):
    T = xT.shape[1]
    jac4 = jnp.zeros((4, T), jnp.float32)
    scale_w = 1.0 - _MIN_BW * _BINS
    scale_h = 1.0 - _MIN_BH * _BINS
    dscale = 1.0 / (_MIN_D + math.log(2.0))

    for i in range(_N_BLOCKS):
        mask = _MASK_TABLE[i]
        cond_idx = [int(v) for v in np.where(mask)[0]]
        trafo_idx = [int(v) for v in np.where(~mask)[0]]

        xc = jnp.concatenate(
            [xT[ci:ci + 1, :] for ci in cond_idx] + [cT], axis=0)
        hh = _dot3(W1hi_ref[i], W1lo_ref[i], xc) + b1_ref[i]
        hh = jnp.where(hh >= 0.0, hh, 0.01 * hh)
        hh = _dot3(W2hi_ref[i], W2lo_ref[i], hh) + b2_ref[i]
        hh = jnp.where(hh >= 0.0, hh, 0.01 * hh)
        so = _dot3(W3hi_ref[i], W3lo_ref[i], hh) + b3_ref[i]

        ew = jnp.exp(jnp.clip(so[0:40], -60.0, 60.0))
        eh = jnp.exp(jnp.clip(so[40:80], -60.0, 60.0))
        ud = so[80:124]

        sw = _ssum40(ew)
        sh = _ssum40(eh)
        ewhi, ewlo = _bsplit(ew)
        ehhi, ehlo = _bsplit(eh)
        cew = (jnp.dot(mstrict, ewhi, preferred_element_type=jnp.float32)
               + jnp.dot(mstrict, ewlo, preferred_element_type=jnp.float32))
        ceh = (jnp.dot(mstrict, ehhi, preferred_element_type=jnp.float32)
               + jnp.dot(mstrict, ehlo, preferred_element_type=jnp.float32))
        bw_in = kvec36 + _tile_rows(scale_w / sw, 9) * cew
        bh_in = kvec36 + _tile_rows(scale_h / sh, 9) * ceh
        zero4 = jnp.zeros((4, T), jnp.float32)
        one4 = jnp.ones((4, T), jnp.float32)
        cumw = jnp.concatenate([zero4, bw_in, one4], axis=0)
        cumh = jnp.concatenate([zero4, bh_in, one4], axis=0)

        deriv = (_MIN_D + _softplus(ud)) * dscale

        x_in = jnp.concatenate(
            [xT[ti:ti + 1, :] for ti in trafo_idx], axis=0)
        ok = ((x_in >= 0.0) & (x_in <= 1.0)).astype(jnp.float32)
        inside = jnp.min(ok, axis=0, keepdims=True) > 0.5
        inp = jnp.clip(x_in, 0.0, 1.0)

        ge = (_tile_rows(inp, 9) >= bw_in).astype(jnp.float32)
        s9 = ge[0:8] + ge[8:16] + ge[16:24] + ge[24:32]
        idx = s9[0:4] + s9[4:8] + ge[32:36]

        onehot = (iota40b == _tile_rows(idx, 10)).astype(jnp.float32)

        widths = cumw[4:] - cumw[:-4]
        heights = cumh[4:] - cumh[:-4]

        in_cw = _ssum40(onehot * cumw[:40])
        in_w = _ssum40(onehot * widths)
        in_ch = _ssum40(onehot * cumh[:40])
        in_h = _ssum40(onehot * heights)
        in_d = _ssum40(onehot * deriv[:40])
        in_dp1 = _ssum40(onehot * deriv[4:])
        in_delta = in_h / in_w

        theta = (inp - in_cw) / in_w
        tomt = theta * (1.0 - theta)
        numer = in_h * (in_delta * theta * theta + in_d * tomt)
        denom = in_delta + (in_d + in_dp1 - 2.0 * in_delta) * tomt
        out = in_ch + numer / denom
        dnum = in_delta * in_delta * (
            in_dp1 * theta * theta + 2.0 * in_delta * tomt
            + in_d * (1.0 - theta) * (1.0 - theta))
        lad = jnp.log(dnum) - 2.0 * jnp.log(denom)

        x_new = jnp.where(inside, out, x_in)
        jac4 = jac4 + jnp.where(inside, lad, 0.0)

        rows = [None] * 8
        for ci in cond_idx:
            rows[ci] = xT[ci:ci + 1]
        for k, ti in enumerate(trafo_idx):
            rows[ti] = x_new[k:k + 1]
        xT = jnp.concatenate(rows, axis=0)

    return xT, jac4


def kernel(x, c, W1, b1, W2, b2, W3, b3):
    B = x.shape[0]
    tile = _TILE if B % _TILE == 0 else B
    grid = (B // tile,)

    xT = x.T
    cT = c.T
    perm = np.asarray(_PERM)
    W1t = jnp.swapaxes(W1, 1, 2)
    W2t = jnp.swapaxes(W2, 1, 2)
    W3t = jnp.swapaxes(W3, 1, 2)[:, perm, :]
    W1hi, W1lo = _bsplit(W1t)
    W2hi, W2lo = _bsplit(W2t)
    W3hi, W3lo = _bsplit(W3t)
    b1e = b1[:, :, None]
    b2e = b2[:, :, None]
    b3e = b3[:, perm, None]

    full = lambda shp: pl.BlockSpec(shp, lambda i: (0,) * len(shp))

    xoT, jac = pl.pallas_call(
        _flow_body,
        grid=grid,
        in_specs=[
            pl.BlockSpec((8, tile), lambda i: (0, i)),
            pl.BlockSpec((4, tile), lambda i: (0, i)),
            full(W1hi.shape), full(W1lo.shape), full(b1e.shape),
            full(W2hi.shape), full(W2lo.shape), full(b2e.shape),
            full(W3hi.shape), full(W3lo.shape), full(b3e.shape),
        ],
        out_specs=[
            pl.BlockSpec((8, tile), lambda i: (0, i)),
            pl.BlockSpec((1, tile), lambda i: (0, i)),
        ],
        out_shape=[
            jax.ShapeDtypeStruct((8, B), jnp.float32),
            jax.ShapeDtypeStruct((1, B), jnp.float32),
        ],
        compiler_params=pltpu.CompilerParams(
            dimension_semantics=("arbitrary",)),
    )(xT, cT, W1hi, W1lo, b1e, W2hi, W2lo, b2e, W3hi, W3lo, b3e)

    return xoT.T, jac[0]


# speedup vs baseline: 1.3641x; 1.3641x over previous
import math
import numpy as np
import jax
import jax.numpy as jnp
from jax.experimental import pallas as pl
from jax.experimental.pallas import tpu as pltpu

_BINS = 10
_N_BLOCKS = 6
_MIN_BW = 1e-3
_MIN_BH = 1e-3
_MIN_D = 1e-3
_TILE = 4096
_PREC = jax.lax.Precision.HIGHEST


def _build_mask_table():
    bits = np.array([[int(b) for b in np.binary_repr(i, 3)] for i in range(8)])
    m = bits[:, ::-1].astype(bool).T
    m = np.repeat(m, 2, axis=0)
    m[1::2] = ~m[1::2]
    return m


_MASK_TABLE = _build_mask_table()

_PERM = ([31 * d + b for b in range(_BINS) for d in range(4)]
         + [31 * d + _BINS + b for b in range(_BINS) for d in range(4)]
         + [31 * d + 2 * _BINS + k for k in range(_BINS + 1) for d in range(4)])


def _ssum40(a):
    s = a[0:8] + a[8:16] + a[16:24] + a[24:32] + a[32:40]
    return s[0:4] + s[4:8]


def _ssum44(a):
    s = a[0:8] + a[8:16] + a[16:24] + a[24:32] + a[32:40]
    return s[0:4] + s[4:8] + a[40:44]


def _tile_rows(a, n):
    a8 = jnp.concatenate([a, a], axis=0)
    reps = [a8] * (n // 2)
    if n % 2:
        reps.append(a)
    return jnp.concatenate(reps, axis=0)


def _softplus(v):
    return jnp.maximum(v, 0.0) + jnp.log1p(jnp.exp(-jnp.abs(v)))


def _bsplit(a):
    hi = a.astype(jnp.bfloat16)
    lo = (a - hi.astype(jnp.float32)).astype(jnp.bfloat16)
    return hi, lo


def _dot3(whi, wlo, act):
    ahi, alo = _bsplit(act)
    r = jnp.dot(whi, ahi, preferred_element_type=jnp.float32)
    r = r + jnp.dot(whi, alo, preferred_element_type=jnp.float32)
    r = r + jnp.dot(wlo, ahi, preferred_element_type=jnp.float32)
    return r


def _flow_body(xT_ref, cT_ref, W1hi_ref, W1lo_ref, b1_ref,
               W2hi_ref, W2lo_ref, b2_ref, W3hi_ref, W3lo_ref, b3_ref,
               xo_ref, jac_ref):
    T = xT_ref.shape[1]

    iota40b = (jax.lax.broadcasted_iota(jnp.int32, (40, T), 0)
               // 4).astype(jnp.float32)
    kvec36 = ((jax.lax.broadcasted_iota(jnp.int32, (36, T), 0)
               // 4).astype(jnp.float32) + 1.0) * _MIN_BW
    rr = jax.lax.broadcasted_iota(jnp.int32, (36, 40), 0)
    cc = jax.lax.broadcasted_iota(jnp.int32, (36, 40), 1)
    mstrict = (((rr % 4) == (cc % 4)) & ((cc // 4) < (rr // 4 + 1))
               ).astype(jnp.bfloat16)

    xT, jac4 = _flow_half(xT_ref[...], cT_ref[...], iota40b, kvec36, mstrict,
                          W1hi_ref, W1lo_ref, b1_ref, W2hi_ref, W2lo_ref,
                          b2_ref, W3hi_ref, W3lo_ref, b3_ref)
    xo_ref[...] = xT
    jac_ref[...] = jnp.sum(jac4, axis=0, keepdims=True)


def _flow_half(xT, cT, iota40b, kvec36, mstrict,
               W1hi_ref, W1lo_ref, b1_ref, W2hi_ref, W2lo_ref, b2_ref,
               W3hi_ref, W3lo_ref, b3_ref):
    T = xT.shape[1]
    jac4 = jnp.zeros((4, T), jnp.float32)
    scale_w = 1.0 - _MIN_BW * _BINS
    scale_h = 1.0 - _MIN_BH * _BINS
    dscale = 1.0 / (_MIN_D + math.log(2.0))

    for i in range(_N_BLOCKS):
        mask = _MASK_TABLE[i]
        cond_idx = [int(v) for v in np.where(mask)[0]]
        trafo_idx = [int(v) for v in np.where(~mask)[0]]

        xc = jnp.concatenate(
            [xT[ci:ci + 1, :] for ci in cond_idx] + [cT], axis=0)
        hh = _dot3(W1hi_ref[i], W1lo_ref[i], xc) + b1_ref[i]
        hh = jnp.where(hh >= 0.0, hh, 0.01 * hh)
        hh = _dot3(W2hi_ref[i], W2lo_ref[i], hh) + b2_ref[i]
        hh = jnp.where(hh >= 0.0, hh, 0.01 * hh)
        so = _dot3(W3hi_ref[i], W3lo_ref[i], hh) + b3_ref[i]

        ew = jnp.exp(jnp.clip(so[0:40], -60.0, 60.0))
        eh = jnp.exp(jnp.clip(so[40:80], -60.0, 60.0))
        ud = so[80:124]

        sw = _ssum40(ew)
        sh = _ssum40(eh)
        ewhi, ewlo = _bsplit(ew)
        ehhi, ehlo = _bsplit(eh)
        cew = (jnp.dot(mstrict, ewhi, preferred_element_type=jnp.float32)
               + jnp.dot(mstrict, ewlo, preferred_element_type=jnp.float32))
        ceh = (jnp.dot(mstrict, ehhi, preferred_element_type=jnp.float32)
               + jnp.dot(mstrict, ehlo, preferred_element_type=jnp.float32))
        bw_in = kvec36 + _tile_rows(scale_w / sw, 9) * cew
        bh_in = kvec36 + _tile_rows(scale_h / sh, 9) * ceh
        zero4 = jnp.zeros((4, T), jnp.float32)
        one4 = jnp.ones((4, T), jnp.float32)
        cumw = jnp.concatenate([zero4, bw_in, one4], axis=0)
        cumh = jnp.concatenate([zero4, bh_in, one4], axis=0)

        deriv = (_MIN_D + _softplus(ud)) * dscale

        x_in = jnp.concatenate(
            [xT[ti:ti + 1, :] for ti in trafo_idx], axis=0)
        ok = ((x_in >= 0.0) & (x_in <= 1.0)).astype(jnp.float32)
        inside = jnp.min(ok, axis=0, keepdims=True) > 0.5
        inp = jnp.clip(x_in, 0.0, 1.0)

        ge = (_tile_rows(inp, 9) >= bw_in).astype(jnp.float32)
        s9 = ge[0:8] + ge[8:16] + ge[16:24] + ge[24:32]
        idx = s9[0:4] + s9[4:8] + ge[32:36]

        onehot = (iota40b == _tile_rows(idx, 10)).astype(jnp.float32)

        widths = cumw[4:] - cumw[:-4]
        heights = cumh[4:] - cumh[:-4]

        in_cw = _ssum40(onehot * cumw[:40])
        in_w = _ssum40(onehot * widths)
        in_ch = _ssum40(onehot * cumh[:40])
        in_h = _ssum40(onehot * heights)
        in_d = _ssum40(onehot * deriv[:40])
        in_dp1 = _ssum40(onehot * deriv[4:])
        in_delta = in_h / in_w

        theta = (inp - in_cw) / in_w
        tomt = theta * (1.0 - theta)
        numer = in_h * (in_delta * theta * theta + in_d * tomt)
        denom = in_delta + (in_d + in_dp1 - 2.0 * in_delta) * tomt
        out = in_ch + numer / denom
        dnum = in_delta * in_delta * (
            in_dp1 * theta * theta + 2.0 * in_delta * tomt
            + in_d * (1.0 - theta) * (1.0 - theta))
        lad = jnp.log(dnum) - 2.0 * jnp.log(denom)

        x_new = jnp.where(inside, out, x_in)
        jac4 = jac4 + jnp.where(inside, lad, 0.0)

        rows = [None] * 8
        for ci in cond_idx:
            rows[ci] = xT[ci:ci + 1]
        for k, ti in enumerate(trafo_idx):
            rows[ti] = x_new[k:k + 1]
        xT = jnp.concatenate(rows, axis=0)

    return xT, jac4


def kernel(x, c, W1, b1, W2, b2, W3, b3):
    B = x.shape[0]
    tile = _TILE if B % _TILE == 0 else B
    grid = (B // tile,)

    xT = x.T
    cT = c.T
    perm = np.asarray(_PERM)
    W1t = jnp.swapaxes(W1, 1, 2)
    W2t = jnp.swapaxes(W2, 1, 2)
    W3t = jnp.swapaxes(W3, 1, 2)[:, perm, :]
    W1hi, W1lo = _bsplit(W1t)
    W2hi, W2lo = _bsplit(W2t)
    W3hi, W3lo = _bsplit(W3t)
    b1e = b1[:, :, None]
    b2e = b2[:, :, None]
    b3e = b3[:, perm, None]

    full = lambda shp: pl.BlockSpec(shp, lambda i: (0,) * len(shp))

    xoT, jac = pl.pallas_call(
        _flow_body,
        grid=grid,
        in_specs=[
            pl.BlockSpec((8, tile), lambda i: (0, i)),
            pl.BlockSpec((4, tile), lambda i: (0, i)),
            full(W1hi.shape), full(W1lo.shape), full(b1e.shape),
            full(W2hi.shape), full(W2lo.shape), full(b2e.shape),
            full(W3hi.shape), full(W3lo.shape), full(b3e.shape),
        ],
        out_specs=[
            pl.BlockSpec((8, tile), lambda i: (0, i)),
            pl.BlockSpec((1, tile), lambda i: (0, i)),
        ],
        out_shape=[
            jax.ShapeDtypeStruct((8, B), jnp.float32),
            jax.ShapeDtypeStruct((1, B), jnp.float32),
        ],
        compiler_params=pltpu.CompilerParams(
            dimension_semantics=("arbitrary",)),
    )(xT, cT, W1hi, W1lo, b1e, W2hi, W2lo, b2e, W3hi, W3lo, b3e)

    return xoT.T, jac[0]
